# baseline (speedup 1.0000x reference)
import functools

import jax
import jax.numpy as jnp
from jax import lax
from jax.experimental import pallas as pl
from jax.experimental.pallas import tpu as pltpu

_E = 1600000
_K = 20
_S = 25000

_TB = 6400
_SB = 64


def _tc_kernel(wb_ref, theta_ref, alpha_ref, label_ref, idx_ref, out_ref,
               acc_ref, vals_ref, *, nsteps, s_real, e_total, k, sb):
    t = pl.program_id(0)

    @pl.when(t == 0)
    def _init():
        acc_ref[...] = jnp.zeros_like(acc_ref)
        vals_ref[2 * k:2 * k + 1, :] = jnp.ones((1, vals_ref.shape[1]),
                                                jnp.float32)

    th = theta_ref[...]
    y = label_ref[0]
    adj = (jnp.maximum(th, 0.0) - th * y
           + jnp.log1p(jnp.exp(-jnp.abs(th))))
    vals_ref[0:k, :] = adj
    vals_ref[k:2 * k, :] = alpha_ref[...]
    vals = vals_ref[...]

    ii = idx_ref[0]
    nq = 4
    qtb = ii.shape[1] // nq
    srow = lax.broadcasted_iota(jnp.int32, (sb, qtb), 0)
    for q in range(nq):
        iq = ii[:, q * qtb:(q + 1) * qtb]
        vq = vals[:, q * qtb:(q + 1) * qtb]
        w_lo = wb_ref[t * nq + q, 0]
        n_w = wb_ref[t * nq + q, 1]

        def body(j, _, iq=iq, vq=vq, w_lo=w_lo):
            w = w_lo + j
            oh = jnp.where(iq - w * sb == srow, 1.0, 0.0)
            part = lax.dot_general(oh, vq, (((1,), (1,)), ((), ())),
                                   preferred_element_type=jnp.float32)
            acc_ref[pl.ds(w * sb, sb), :] += part
            return 0

        lax.fori_loop(0, n_w, body, 0)

    @pl.when(t == nsteps - 1)
    def _epilogue():
        sp = acc_ref.shape[0]
        chunk = next((c_ for c_ in range(1024, 7, -8) if sp % c_ == 0), sp)
        nchunk = -(-sp // chunk)

        def ebody(i, acc_s):
            a = acc_ref[pl.ds(i * chunk, chunk), :]
            ra = a[:, 0:k]
            cnt = a[:, 2 * k:2 * k + 1]
            la = a[:, k:2 * k] / jnp.maximum(cnt, 1.0)
            m1 = jnp.max(la, axis=1, keepdims=True)
            lse1 = m1 + jnp.log(jnp.sum(jnp.exp(la - m1), axis=1,
                                        keepdims=True))
            lp = -ra + (la - lse1)
            m2 = jnp.max(lp, axis=1, keepdims=True)
            lpe = m2 + jnp.log(jnp.sum(jnp.exp(lp - m2), axis=1,
                                       keepdims=True))
            row = lax.broadcasted_iota(jnp.int32, lpe.shape, 0) + i * chunk
            lpe = jnp.where(row < s_real, lpe, 0.0)
            return acc_s + jnp.sum(lpe)

        total = lax.fori_loop(0, nchunk, ebody, 0.0)
        out_ref[...] = jnp.full((1, 1), total * (-1.0 / e_total),
                                dtype=jnp.float32)


@functools.partial(jax.jit, static_argnames=("e", "k", "s", "tb", "sb"))
def _run(label, log_theta, log_alpha, subgraph_idx,
         e=_E, k=_K, s=_S, tb=_TB, sb=_SB):
    nsteps = e // tb
    n_windows = -(-s // sb)
    sp = n_windows * sb
    c = 2 * k + 1
    idx_r = subgraph_idx.reshape(nsteps * 4, tb // 4)
    w_lo_all = idx_r[:, 0] // sb
    w_hi_all = idx_r[:, -1] // sb
    wb = jnp.stack([w_lo_all, w_hi_all - w_lo_all + 1], axis=1)
    out = pl.pallas_call(
        functools.partial(_tc_kernel, nsteps=nsteps,
                          s_real=s, e_total=float(e), k=k, sb=sb),
        grid=(nsteps,),
        in_specs=[
            pl.BlockSpec(memory_space=pltpu.SMEM),
            pl.BlockSpec((k, tb), lambda t: (0, t)),
            pl.BlockSpec((k, tb), lambda t: (0, t)),
            pl.BlockSpec((1, 1, tb), lambda t: (t, 0, 0)),
            pl.BlockSpec((1, 1, tb), lambda t: (t, 0, 0)),
        ],
        out_specs=pl.BlockSpec((1, 1), lambda t: (0, 0)),
        out_shape=jax.ShapeDtypeStruct((1, 1), jnp.float32),
        scratch_shapes=[
            pltpu.VMEM((sp, c), jnp.float32),
            pltpu.VMEM((c, tb), jnp.float32),
        ],
    )(wb, log_theta.T, log_alpha.T,
      label.reshape(nsteps, 1, tb), subgraph_idx.reshape(nsteps, 1, tb))
    return out[0, 0]


def kernel(label, log_theta, log_alpha, subgraph_idx):
    return _run(label, log_theta, log_alpha, subgraph_idx)


# speedup vs baseline: 1.2165x; 1.2165x over previous
import functools

import jax
import jax.numpy as jnp
from jax import lax
from jax.experimental import pallas as pl
from jax.experimental.pallas import tpu as pltpu

_E = 1600000
_K = 20
_S = 25000

_TB = 6400
_SB = 64


def _tc_kernel(wb_ref, theta_ref, alpha_ref, label_ref, idx_ref, out_ref,
               acc_ref, vals_ref, *, nsteps, s_real, e_total, k, sb):
    t = pl.program_id(0)

    @pl.when(t == 0)
    def _init():
        acc_ref[...] = jnp.zeros_like(acc_ref)
        vals_ref[2 * k:2 * k + 1, :] = jnp.ones((1, vals_ref.shape[1]),
                                                jnp.bfloat16)

    th = theta_ref[...]
    y = label_ref[0]
    adj = (jnp.maximum(th, 0.0) - th * y
           + jnp.log1p(jnp.exp(-jnp.abs(th))))
    vals_ref[0:k, :] = adj.astype(jnp.bfloat16)
    vals_ref[k:2 * k, :] = alpha_ref[...].astype(jnp.bfloat16)
    vals = vals_ref[...]

    ii = idx_ref[0]
    w_lo = wb_ref[t, 0]
    n_w = wb_ref[t, 1]

    srow = lax.broadcasted_iota(jnp.int32, (sb, ii.shape[1]), 0)

    def body(j, _):
        w = w_lo + j
        oh = jnp.where(ii - w * sb == srow,
                       1.0, 0.0).astype(jnp.bfloat16)
        part = lax.dot_general(oh, vals, (((1,), (1,)), ((), ())),
                               preferred_element_type=jnp.float32)
        acc_ref[pl.ds(w * sb, sb), :] += part
        return 0

    lax.fori_loop(0, n_w, body, 0)

    @pl.when(t == nsteps - 1)
    def _epilogue():
        sp = acc_ref.shape[0]
        chunk = next((c_ for c_ in range(1024, 7, -8) if sp % c_ == 0), sp)
        nchunk = -(-sp // chunk)

        def ebody(i, acc_s):
            a = acc_ref[pl.ds(i * chunk, chunk), :]
            ra = a[:, 0:k]
            cnt = a[:, 2 * k:2 * k + 1]
            la = a[:, k:2 * k] / jnp.maximum(cnt, 1.0)
            m1 = jnp.max(la, axis=1, keepdims=True)
            lse1 = m1 + jnp.log(jnp.sum(jnp.exp(la - m1), axis=1,
                                        keepdims=True))
            lp = -ra + (la - lse1)
            m2 = jnp.max(lp, axis=1, keepdims=True)
            lpe = m2 + jnp.log(jnp.sum(jnp.exp(lp - m2), axis=1,
                                       keepdims=True))
            row = lax.broadcasted_iota(jnp.int32, lpe.shape, 0) + i * chunk
            lpe = jnp.where(row < s_real, lpe, 0.0)
            return acc_s + jnp.sum(lpe)

        total = lax.fori_loop(0, nchunk, ebody, 0.0)
        out_ref[...] = jnp.full((1, 1), total * (-1.0 / e_total),
                                dtype=jnp.float32)


@functools.partial(jax.jit, static_argnames=("e", "k", "s", "tb", "sb"))
def _run(label, log_theta, log_alpha, subgraph_idx,
         e=_E, k=_K, s=_S, tb=_TB, sb=_SB):
    nsteps = e // tb
    n_windows = -(-s // sb)
    sp = n_windows * sb
    c = 2 * k + 1
    idx_r = subgraph_idx.reshape(nsteps, tb)
    w_lo_all = idx_r[:, 0] // sb
    w_hi_all = idx_r[:, -1] // sb
    wb = jnp.stack([w_lo_all, w_hi_all - w_lo_all + 1], axis=1)
    out = pl.pallas_call(
        functools.partial(_tc_kernel, nsteps=nsteps,
                          s_real=s, e_total=float(e), k=k, sb=sb),
        grid=(nsteps,),
        in_specs=[
            pl.BlockSpec(memory_space=pltpu.SMEM),
            pl.BlockSpec((k, tb), lambda t: (0, t)),
            pl.BlockSpec((k, tb), lambda t: (0, t)),
            pl.BlockSpec((1, 1, tb), lambda t: (t, 0, 0)),
            pl.BlockSpec((1, 1, tb), lambda t: (t, 0, 0)),
        ],
        out_specs=pl.BlockSpec((1, 1), lambda t: (0, 0)),
        out_shape=jax.ShapeDtypeStruct((1, 1), jnp.float32),
        scratch_shapes=[
            pltpu.VMEM((sp, c), jnp.float32),
            pltpu.VMEM((c, tb), jnp.bfloat16),
        ],
    )(wb, log_theta.T, log_alpha.T,
      label.reshape(nsteps, 1, tb), subgraph_idx.reshape(nsteps, 1, tb))
    return out[0, 0]


def kernel(label, log_theta, log_alpha, subgraph_idx):
    return _run(label, log_theta, log_alpha, subgraph_idx)
